# Initial kernel scaffold; baseline (speedup 1.0000x reference)
import jax
import jax.numpy as jnp
from jax.experimental import pallas as pl

_E = 64
_K = 8
_BT = 2048


def _gating_kernel(x_ref, w_ref, b_ref, vals_ref, idx_ref):
    x = x_ref[...]
    w = w_ref[...]
    logits32 = jax.lax.dot_general(
        x, w, (((1,), (1,)), ((), ())), preferred_element_type=jnp.float32
    )
    logits = logits32.astype(jnp.float16) + b_ref[0:1, :]

    m = jnp.max(logits, axis=-1, keepdims=True)
    e = jnp.exp(logits - m)
    s = jnp.sum(e, axis=-1, keepdims=True).astype(jnp.float16)
    scores = e / s

    bits = jax.lax.bitcast_convert_type(scores, jnp.uint16).astype(jnp.int32)
    lane = jax.lax.broadcasted_iota(jnp.int32, scores.shape, 1)
    keys = (bits << 6) | (_E - 1 - lane)

    top_vals = []
    top_idx = []
    for _ in range(_K):
        mk = jnp.max(keys, axis=-1, keepdims=True)
        top_idx.append(_E - 1 - (mk & (_E - 1)))
        vb = (mk >> 6).astype(jnp.uint16)
        top_vals.append(jax.lax.bitcast_convert_type(vb, jnp.float16))
        keys = jnp.where(keys == mk, -1, keys)

    v = jnp.concatenate(top_vals, axis=-1).astype(jnp.float32)
    i = jnp.concatenate(top_idx, axis=-1)

    e2 = jnp.exp(v - v[:, 0:1])
    out = e2 / jnp.sum(e2, axis=-1, keepdims=True)
    vals_ref[...] = out.astype(jnp.float16)
    idx_ref[...] = i


@jax.jit
def kernel(x, W, b):
    T, D = x.shape
    E = W.shape[0]
    b2 = jnp.broadcast_to(b.reshape(1, E), (16, E))
    grid = (T // _BT,)
    vals, idx = pl.pallas_call(
        _gating_kernel,
        grid=grid,
        in_specs=[
            pl.BlockSpec((_BT, D), lambda t: (t, 0)),
            pl.BlockSpec((E, D), lambda t: (0, 0)),
            pl.BlockSpec((16, E), lambda t: (0, 0)),
        ],
        out_specs=[
            pl.BlockSpec((_BT, _K), lambda t: (t, 0)),
            pl.BlockSpec((_BT, _K), lambda t: (t, 0)),
        ],
        out_shape=[
            jax.ShapeDtypeStruct((T, _K), jnp.float16),
            jax.ShapeDtypeStruct((T, _K), jnp.int32),
        ],
    )(x, W, b2)
    return vals, idx


# speedup vs baseline: 1.3226x; 1.3226x over previous
import jax
import jax.numpy as jnp
from jax.experimental import pallas as pl

_E = 64
_K = 8
_BT = 2048

_F16_SUBNORM = 6.103515625e-05
_TWO24 = 16777216.0
_INV_TWO24 = 5.960464477539063e-08
_MAGIC = 12582912.0


def _round_f16(v):
    b = jax.lax.bitcast_convert_type(v, jnp.int32)
    rb = (b + 0x0FFF + ((b >> 13) & 1)) & ~0x1FFF
    vn = jax.lax.bitcast_convert_type(rb, jnp.float32)
    u = v * _TWO24
    vs = ((u + _MAGIC) - _MAGIC) * _INV_TWO24
    return jnp.where(jnp.abs(v) < _F16_SUBNORM, vs, vn)


def _gating_kernel(x_ref, w_ref, b_ref, vals_ref, idx_ref):
    logits32 = jax.lax.dot_general(
        x_ref[...], w_ref[...], (((1,), (1,)), ((), ())),
        preferred_element_type=jnp.float32
    )

    lm = _round_f16(logits32 + b_ref[0:1, :])
    m = jnp.max(lm, axis=-1, keepdims=True)
    e = jnp.exp(lm - m)
    s = jnp.sum(e, axis=-1, keepdims=True)
    scores = _round_f16(e / s)

    sbits = jax.lax.bitcast_convert_type(scores, jnp.int32)
    lane = jax.lax.broadcasted_iota(jnp.int32, scores.shape, 1)
    keys = (((sbits >> 13) << 6) | (_E - 1 - lane)).astype(jnp.float32)

    top_vals = []
    top_idx = []
    for _ in range(_K):
        mk = jnp.max(keys, axis=-1, keepdims=True)
        mki = mk.astype(jnp.int32)
        top_idx.append(_E - 1 - (mki & (_E - 1)))
        top_vals.append(jax.lax.bitcast_convert_type(
            (mki >> 6) << 13, jnp.float32))
        keys = jnp.where(keys == mk, -1.0, keys)

    v = jnp.concatenate(top_vals, axis=-1)
    i = jnp.concatenate(top_idx, axis=-1)

    e2 = jnp.exp(v - v[:, 0:1])
    out = e2 / jnp.sum(e2, axis=-1, keepdims=True)
    vals_ref[...] = out
    idx_ref[...] = i


@jax.jit
def kernel(x, W, b):
    T, D = x.shape
    E = W.shape[0]
    xb = x.astype(jnp.bfloat16)
    wb = W.astype(jnp.bfloat16)
    b2 = jnp.broadcast_to(b.astype(jnp.float32).reshape(1, E), (8, E))
    grid = (T // _BT,)
    vals, idx = pl.pallas_call(
        _gating_kernel,
        grid=grid,
        in_specs=[
            pl.BlockSpec((_BT, D), lambda t: (t, 0)),
            pl.BlockSpec((E, D), lambda t: (0, 0)),
            pl.BlockSpec((8, E), lambda t: (0, 0)),
        ],
        out_specs=[
            pl.BlockSpec((_BT, _K), lambda t: (t, 0)),
            pl.BlockSpec((_BT, _K), lambda t: (t, 0)),
        ],
        out_shape=[
            jax.ShapeDtypeStruct((T, _K), jnp.float32),
            jax.ShapeDtypeStruct((T, _K), jnp.int32),
        ],
    )(xb, wb, b2)
    return vals.astype(jnp.float16), idx


# speedup vs baseline: 1.4901x; 1.1267x over previous
import jax
import jax.numpy as jnp
from jax.experimental import pallas as pl

_E = 64
_K = 8
_BT = 2048

_F16_SUBNORM = 6.103515625e-05
_TWO24 = 16777216.0
_INV_TWO24 = 5.960464477539063e-08
_MAGIC = 12582912.0
_MAGIC2 = 12582912.0 - 114688.0
_RANK_MIN_NORMAL = 115712


def _round_f16_normal(v):
    b = jax.lax.bitcast_convert_type(v, jnp.int32)
    rb = (b + 0x0FFF + ((b >> 13) & 1)) & ~0x1FFF
    return jax.lax.bitcast_convert_type(rb, jnp.float32)


def _score_keys(q, neg_lane):
    b = jax.lax.bitcast_convert_type(q, jnp.int32)
    kn = (b + 0x0FFF + ((b >> 13) & 1)) >> 13
    ks = ((q * _TWO24 + _MAGIC) - _MAGIC2).astype(jnp.int32)
    kv = jnp.where(q < _F16_SUBNORM, ks, kn)
    return ((kv << 6) | neg_lane).astype(jnp.float32)


def _gating_kernel(x_ref, w_ref, b_ref, vals_ref, idx_ref):
    logits32 = jax.lax.dot_general(
        x_ref[...], w_ref[...], (((1,), (1,)), ((), ())),
        preferred_element_type=jnp.float32
    )

    lm = _round_f16_normal(logits32 + b_ref[0:1, :])
    m = jnp.max(lm, axis=-1, keepdims=True)
    e = jnp.exp(lm - m)
    q = e / jnp.sum(e, axis=-1, keepdims=True)

    lane = jax.lax.broadcasted_iota(jnp.int32, q.shape, 1)
    keys = _score_keys(q, _E - 1 - lane)

    tops = []
    for _ in range(_K):
        mk = jnp.max(keys, axis=-1, keepdims=True)
        tops.append(mk)
        keys = jnp.where(keys == mk, -1.0, keys)

    k8 = jnp.concatenate(tops, axis=-1).astype(jnp.int32)
    i = (_E - 1) - (k8 & (_E - 1))
    kv = k8 >> 6
    v_norm = jax.lax.bitcast_convert_type(kv << 13, jnp.float32)
    v_sub = (kv - 114688).astype(jnp.float32) * _INV_TWO24
    v = jnp.where(kv < _RANK_MIN_NORMAL, v_sub, v_norm)

    e2 = jnp.exp(v - v[:, 0:1])
    out = e2 / jnp.sum(e2, axis=-1, keepdims=True)
    vals_ref[...] = out
    idx_ref[...] = i


@jax.jit
def kernel(x, W, b):
    T, D = x.shape
    E = W.shape[0]
    xb = x.astype(jnp.bfloat16)
    wb = W.astype(jnp.bfloat16)
    b2 = jnp.broadcast_to(b.astype(jnp.float32).reshape(1, E), (8, E))
    grid = (T // _BT,)
    vals, idx = pl.pallas_call(
        _gating_kernel,
        grid=grid,
        in_specs=[
            pl.BlockSpec((_BT, D), lambda t: (t, 0)),
            pl.BlockSpec((E, D), lambda t: (0, 0)),
            pl.BlockSpec((8, E), lambda t: (0, 0)),
        ],
        out_specs=[
            pl.BlockSpec((_BT, _K), lambda t: (t, 0)),
            pl.BlockSpec((_BT, _K), lambda t: (t, 0)),
        ],
        out_shape=[
            jax.ShapeDtypeStruct((T, _K), jnp.float32),
            jax.ShapeDtypeStruct((T, _K), jnp.int32),
        ],
    )(xb, wb, b2)
    return vals.astype(jnp.float16), idx
